# baseline (speedup 1.0000x reference)
import jax
import jax.numpy as jnp
from jax.experimental import pallas as pl
from jax.experimental.pallas import tpu as pltpu

_BAND = 121
_VMEM = 60 * 1024 * 1024


def _conv_block_kernel(p_ref, w1_ref, b1_ref, w2_ref, b2_ref, w3_ref,
                       b3_ref, a3_ref):
    m = p_ref.shape[0]

    xp = jnp.concatenate([p_ref[...], jnp.zeros((16, 256), jnp.bfloat16)],
                         axis=0)
    x1 = jnp.concatenate([xp[off:off + m] for off in (0, 1, 11, 12)], axis=1)
    a1 = jnp.dot(x1, w1_ref[...], preferred_element_type=jnp.float32)
    a1 = jnp.maximum(a1 + b1_ref[...], 0.0).astype(jnp.bfloat16)

    a1p = jnp.concatenate([a1, jnp.zeros((16, 128), jnp.bfloat16)], axis=0)
    x2 = jnp.concatenate([a1p[off:off + m] for off in (0, 1, 11, 12)], axis=1)
    a2 = jnp.dot(x2, w2_ref[...], preferred_element_type=jnp.float32)
    a2 = jnp.maximum(a2 + b2_ref[...], 0.0).astype(jnp.bfloat16)

    a2p = jnp.concatenate([a2, jnp.zeros((24, 64), jnp.bfloat16)], axis=0)
    x3 = jnp.concatenate(
        [a2p[off:off + m] for off in (0, 1, 2, 11, 12, 13, 22, 23, 24)],
        axis=1)
    a3 = jnp.dot(x3, w3_ref[...], preferred_element_type=jnp.float32)
    a3_ref[...] = jnp.maximum(a3 + b3_ref[...], 0.0).astype(jnp.bfloat16)


def _fc_block_kernel(a3_ref, wf1_ref, bf1_ref, wf2_ref, bf2_ref, o_ref):
    a3r = a3_ref[...]
    xf = jnp.concatenate(
        [a3r[:, u * 11 + v, :] for u in range(7) for v in range(7)], axis=1)
    h = jnp.dot(xf, wf1_ref[...], preferred_element_type=jnp.float32)
    h = jnp.maximum(h + bf1_ref[...], 0.0)
    logit = jnp.sum(h * wf2_ref[...], axis=1, keepdims=True) + bf2_ref[...]
    o_ref[...] = pl.reciprocal(1.0 + jnp.exp(-logit), approx=True)


def _s2d_rows(x_nchw):
    b, c, h, w = x_nchw.shape
    xp = jnp.pad(x_nchw, ((0, 0), (0, 0), (0, 88 - h), (0, 88 - w)))
    xp = xp.astype(jnp.bfloat16).reshape(b, c, 11, 8, 11, 8)
    xp = jnp.transpose(xp, (0, 2, 4, 1, 3, 5))
    return xp.reshape(b * _BAND, c * 64)


def _conv1_tap_weights(w1, c):
    t = w1.reshape(8, 8, c, 32)
    subs = []
    for a in (0, 1):
        for bb in (0, 1):
            pad = jnp.pad(t, ((4 * a, 8 - 4 * a), (4 * bb, 8 - 4 * bb),
                              (0, 0), (0, 0)))
            subs.append(pad.reshape(2, 8, 2, 8, c, 32))
    s = jnp.stack(subs, axis=0).reshape(2, 2, 2, 8, 2, 8, c, 32)
    s = jnp.transpose(s, (2, 4, 6, 3, 5, 0, 1, 7))
    return s.reshape(4 * c * 64, 128)


def kernel(w1, b1, w2, b2, w3, b3, wf1, bf1, wf2, bf2, x):
    b = x.shape[0]
    c = x.shape[1]
    img = next(g for g in (32, 16, 8, 4, 2, 1) if b % g == 0)
    img2 = next(g for g in (128, 64, 32, 16, 8, 4, 2, 1) if b % g == 0)

    rows = _s2d_rows(x)
    w1c = _conv1_tap_weights(w1, c)
    b1t = jnp.tile(b1, (1, 4))
    w2c = w2.reshape(4 * 128, 64)
    w3c = w3.reshape(9 * 64, 64)
    wf1c = wf1.reshape(49 * 64, 512)

    m = img * _BAND
    a3 = pl.pallas_call(
        _conv_block_kernel,
        out_shape=jax.ShapeDtypeStruct((b * _BAND, 64), jnp.bfloat16),
        grid=(b // img,),
        in_specs=[
            pl.BlockSpec((m, 256), lambda i: (i, 0)),
            pl.BlockSpec((1024, 128), lambda i: (0, 0)),
            pl.BlockSpec((1, 128), lambda i: (0, 0)),
            pl.BlockSpec((512, 64), lambda i: (0, 0)),
            pl.BlockSpec((1, 64), lambda i: (0, 0)),
            pl.BlockSpec((576, 64), lambda i: (0, 0)),
            pl.BlockSpec((1, 64), lambda i: (0, 0)),
        ],
        out_specs=pl.BlockSpec((m, 64), lambda i: (i, 0)),
        compiler_params=pltpu.CompilerParams(
            dimension_semantics=("parallel",),
            vmem_limit_bytes=_VMEM),
    )(rows, w1c, b1t, w2c, b2, w3c, b3)
    a3 = a3.reshape(b, _BAND, 64)

    out = pl.pallas_call(
        _fc_block_kernel,
        out_shape=jax.ShapeDtypeStruct((b, 1), jnp.float32),
        grid=(b // img2,),
        in_specs=[
            pl.BlockSpec((img2, _BAND, 64), lambda i: (i, 0, 0)),
            pl.BlockSpec((3136, 512), lambda i: (0, 0)),
            pl.BlockSpec((1, 512), lambda i: (0, 0)),
            pl.BlockSpec((1, 512), lambda i: (0, 0)),
            pl.BlockSpec((1, 1), lambda i: (0, 0)),
        ],
        out_specs=pl.BlockSpec((img2, 1), lambda i: (i, 0)),
        compiler_params=pltpu.CompilerParams(
            dimension_semantics=("parallel",),
            vmem_limit_bytes=_VMEM),
    )(a3, wf1c, bf1, wf2, bf2)
    return out


# speedup vs baseline: 2.5447x; 2.5447x over previous
import jax
import jax.numpy as jnp
from jax.experimental import pallas as pl
from jax.experimental.pallas import tpu as pltpu

_BAND = 128
_VMEM = 60 * 1024 * 1024


def _fused_block_kernel(p_ref, w1_ref, b1_ref, w2_ref, b2_ref, w3_ref,
                        b3_ref, wf1_ref, bf1_ref, wf2_ref, bf2_ref, o_ref):
    m = p_ref.shape[0]
    img = m // _BAND

    xp = jnp.concatenate([p_ref[...], jnp.zeros((16, 256), jnp.bfloat16)],
                         axis=0)
    x1 = jnp.concatenate([xp[off:off + m] for off in (0, 1, 11, 12)], axis=1)
    a1 = jnp.dot(x1, w1_ref[...], preferred_element_type=jnp.float32)
    a1 = jnp.maximum(a1 + b1_ref[...], 0.0).astype(jnp.bfloat16)

    a1p = jnp.concatenate([a1, jnp.zeros((16, 128), jnp.bfloat16)], axis=0)
    x2 = jnp.concatenate([a1p[off:off + m] for off in (0, 1, 11, 12)], axis=1)
    a2 = jnp.dot(x2, w2_ref[...], preferred_element_type=jnp.float32)
    a2 = jnp.maximum(a2 + b2_ref[...], 0.0).astype(jnp.bfloat16)

    a2p = jnp.concatenate([a2, jnp.zeros((24, 64), jnp.bfloat16)], axis=0)
    x3 = jnp.concatenate(
        [a2p[off:off + m] for off in (0, 1, 2, 11, 12, 13, 22, 23, 24)],
        axis=1)
    a3 = jnp.dot(x3, w3_ref[...], preferred_element_type=jnp.float32)
    a3 = jnp.maximum(a3 + b3_ref[...], 0.0).astype(jnp.bfloat16)

    a3r = a3.reshape(img, _BAND, 64)
    xf = jnp.concatenate(
        [a3r[:, u * 11 + v, :] for u in range(7) for v in range(7)], axis=1)
    h = jnp.dot(xf, wf1_ref[...], preferred_element_type=jnp.float32)
    h = jnp.maximum(h + bf1_ref[...], 0.0)

    logit = jnp.sum(h * wf2_ref[...], axis=1, keepdims=True) + bf2_ref[...]
    o_ref[...] = pl.reciprocal(1.0 + jnp.exp(-logit), approx=True)


def _s2d_rows(x_nchw):
    b, c, h, w = x_nchw.shape
    xp = jnp.pad(x_nchw, ((0, 0), (0, 0), (0, 88 - h), (0, 88 - w)))
    xp = xp.astype(jnp.bfloat16).reshape(b, c, 11, 8, 11, 8)
    xp = jnp.transpose(xp, (0, 2, 4, 1, 3, 5)).reshape(b, 121, c * 64)
    xp = jnp.pad(xp, ((0, 0), (0, _BAND - 121), (0, 0)))
    return xp.reshape(b * _BAND, c * 64)


def _conv1_tap_weights(w1, c):
    t = w1.reshape(8, 8, c, 32)
    subs = []
    for a in (0, 1):
        for bb in (0, 1):
            pad = jnp.pad(t, ((4 * a, 8 - 4 * a), (4 * bb, 8 - 4 * bb),
                              (0, 0), (0, 0)))
            subs.append(pad.reshape(2, 8, 2, 8, c, 32))
    s = jnp.stack(subs, axis=0).reshape(2, 2, 2, 8, 2, 8, c, 32)
    s = jnp.transpose(s, (2, 4, 6, 3, 5, 0, 1, 7))
    return s.reshape(4 * c * 64, 128)


def kernel(w1, b1, w2, b2, w3, b3, wf1, bf1, wf2, bf2, x):
    b = x.shape[0]
    c = x.shape[1]
    img = next(g for g in (16, 8, 4, 2, 1) if b % g == 0)

    rows = _s2d_rows(x)
    w1c = _conv1_tap_weights(w1, c)
    b1t = jnp.tile(b1, (1, 4))
    w2c = w2.reshape(4 * 128, 64)
    w3c = w3.reshape(9 * 64, 64)
    wf1c = wf1.reshape(49 * 64, 512)

    m = img * _BAND
    out = pl.pallas_call(
        _fused_block_kernel,
        out_shape=jax.ShapeDtypeStruct((b, 1), jnp.float32),
        grid=(b // img,),
        in_specs=[
            pl.BlockSpec((m, 256), lambda i: (i, 0)),
            pl.BlockSpec((1024, 128), lambda i: (0, 0)),
            pl.BlockSpec((1, 128), lambda i: (0, 0)),
            pl.BlockSpec((512, 64), lambda i: (0, 0)),
            pl.BlockSpec((1, 64), lambda i: (0, 0)),
            pl.BlockSpec((576, 64), lambda i: (0, 0)),
            pl.BlockSpec((1, 64), lambda i: (0, 0)),
            pl.BlockSpec((3136, 512), lambda i: (0, 0)),
            pl.BlockSpec((1, 512), lambda i: (0, 0)),
            pl.BlockSpec((1, 512), lambda i: (0, 0)),
            pl.BlockSpec((1, 1), lambda i: (0, 0)),
        ],
        out_specs=pl.BlockSpec((img, 1), lambda i: (i, 0)),
        compiler_params=pltpu.CompilerParams(
            dimension_semantics=("parallel",),
            vmem_limit_bytes=_VMEM),
    )(rows, w1c, b1t, w2c, b2, w3c, b3, wf1c, bf1, wf2, bf2)
    return out


# speedup vs baseline: 2.6378x; 1.0366x over previous
import jax
import jax.numpy as jnp
from jax.experimental import pallas as pl
from jax.experimental.pallas import tpu as pltpu

_BAND = 128
_VMEM = 60 * 1024 * 1024


def _fused_block_kernel(p_ref, w1_ref, b1_ref, w2_ref, b2_ref, w3_ref,
                        b3_ref, wf1_ref, bf1_ref, wf2_ref, bf2_ref, o_ref):
    m = p_ref.shape[0]
    img = m // _BAND

    xp = jnp.concatenate([p_ref[...], jnp.zeros((16, 256), jnp.bfloat16)],
                         axis=0)
    x1 = jnp.concatenate([xp[off:off + m] for off in (0, 1, 11, 12)], axis=1)
    a1 = jnp.dot(x1, w1_ref[...], preferred_element_type=jnp.float32)
    a1 = jnp.maximum(a1 + b1_ref[...], 0.0).astype(jnp.bfloat16)

    a1p = jnp.concatenate([a1, jnp.zeros((16, 128), jnp.bfloat16)], axis=0)
    x2 = jnp.concatenate([a1p[off:off + m] for off in (0, 1, 11, 12)], axis=1)
    a2 = jnp.dot(x2, w2_ref[...], preferred_element_type=jnp.float32)
    a2 = jnp.maximum(a2 + b2_ref[...], 0.0).astype(jnp.bfloat16)

    a2p = jnp.concatenate([a2, jnp.zeros((24, 64), jnp.bfloat16)], axis=0)
    x3 = jnp.concatenate(
        [a2p[off:off + m] for off in (0, 1, 2, 11, 12, 13, 22, 23, 24)],
        axis=1)
    a3 = jnp.dot(x3, w3_ref[...], preferred_element_type=jnp.float32)
    a3 = jnp.maximum(a3 + b3_ref[...], 0.0).astype(jnp.bfloat16)

    a3r = a3.reshape(img, _BAND, 64)
    xf = jnp.concatenate(
        [a3r[:, u * 11 + v, :] for u in range(7) for v in range(7)], axis=1)
    h = jnp.dot(xf, wf1_ref[...], preferred_element_type=jnp.float32)
    h = jnp.maximum(h + bf1_ref[...], 0.0)

    logit = jnp.sum(h * wf2_ref[...], axis=1, keepdims=True) + bf2_ref[...]
    o_ref[...] = pl.reciprocal(1.0 + jnp.exp(-logit), approx=True)


def _s2d_rows(x_nchw):
    b, c, h, w = x_nchw.shape
    xp = jnp.pad(x_nchw, ((0, 0), (0, 0), (0, 88 - h), (0, 88 - w)))
    xp = xp.astype(jnp.bfloat16).reshape(b, c, 11, 8, 11, 8)
    xp = jnp.transpose(xp, (0, 2, 4, 1, 3, 5)).reshape(b, 121, c * 64)
    xp = jnp.pad(xp, ((0, 0), (0, _BAND - 121), (0, 0)))
    return xp.reshape(b * _BAND, c * 64)


def _conv1_tap_weights(w1, c):
    t = w1.reshape(8, 8, c, 32)
    subs = []
    for a in (0, 1):
        for bb in (0, 1):
            pad = jnp.pad(t, ((4 * a, 8 - 4 * a), (4 * bb, 8 - 4 * bb),
                              (0, 0), (0, 0)))
            subs.append(pad.reshape(2, 8, 2, 8, c, 32))
    s = jnp.stack(subs, axis=0).reshape(2, 2, 2, 8, 2, 8, c, 32)
    s = jnp.transpose(s, (2, 4, 6, 3, 5, 0, 1, 7))
    return s.reshape(4 * c * 64, 128)


def kernel(w1, b1, w2, b2, w3, b3, wf1, bf1, wf2, bf2, x):
    b = x.shape[0]
    c = x.shape[1]
    img = next(g for g in (64, 32, 16, 8, 4, 2, 1) if b % g == 0)

    rows = _s2d_rows(x)
    w1c = _conv1_tap_weights(w1, c)
    b1t = jnp.tile(b1, (1, 4))
    w2c = w2.reshape(4 * 128, 64)
    w3c = w3.reshape(9 * 64, 64)
    wf1c = wf1.reshape(49 * 64, 512)

    m = img * _BAND
    out = pl.pallas_call(
        _fused_block_kernel,
        out_shape=jax.ShapeDtypeStruct((b, 1), jnp.float32),
        grid=(b // img,),
        in_specs=[
            pl.BlockSpec((m, 256), lambda i: (i, 0)),
            pl.BlockSpec((1024, 128), lambda i: (0, 0)),
            pl.BlockSpec((1, 128), lambda i: (0, 0)),
            pl.BlockSpec((512, 64), lambda i: (0, 0)),
            pl.BlockSpec((1, 64), lambda i: (0, 0)),
            pl.BlockSpec((576, 64), lambda i: (0, 0)),
            pl.BlockSpec((1, 64), lambda i: (0, 0)),
            pl.BlockSpec((3136, 512), lambda i: (0, 0)),
            pl.BlockSpec((1, 512), lambda i: (0, 0)),
            pl.BlockSpec((1, 512), lambda i: (0, 0)),
            pl.BlockSpec((1, 1), lambda i: (0, 0)),
        ],
        out_specs=pl.BlockSpec((img, 1), lambda i: (i, 0)),
        compiler_params=pltpu.CompilerParams(
            dimension_semantics=("parallel",),
            vmem_limit_bytes=_VMEM),
    )(rows, w1c, b1t, w2c, b2, w3c, b3, wf1c, bf1, wf2, bf2)
    return out
